# Initial kernel scaffold; baseline (speedup 1.0000x reference)
import jax
import jax.numpy as jnp
from jax.experimental import pallas as pl


def kernel(x, edge_index, edge_attr, batch, macro_metrics, W1, b1, macro_mean, macro_std):
    raise NotImplementedError("write your pallas kernel here")


# speedup vs baseline: 6.6271x; 6.6271x over previous
import functools

import jax
import jax.numpy as jnp
from jax import lax
from jax.experimental import pallas as pl
from jax.experimental.pallas import tpu as pltpu
from jax.experimental.pallas import tpu_sc as plsc

_SIGMA = 1.0
_INV_S = 1.0 / (_SIGMA**2 + 1e-06)
_BIG = 3.0e38
_L = 16


def _mm_body(x_ref, w_ref, o_ref):
    o_ref[...] = jnp.dot(x_ref[...], w_ref[...],
                         preferred_element_type=jnp.float32,
                         precision=lax.Precision.HIGHEST)


def _matmul(x, w, nb):
    n, d = x.shape
    blk = n // nb
    return pl.pallas_call(
        _mm_body,
        grid=(nb,),
        in_specs=[
            pl.BlockSpec((blk, d), lambda i: (i, 0)),
            pl.BlockSpec((d, d), lambda i: (0, 0)),
        ],
        out_specs=pl.BlockSpec((blk, d), lambda i: (i, 0)),
        out_shape=jax.ShapeDtypeStruct((n, d), jnp.float32),
    )(x, w)


def _make_sc_min(n_nodes, n_edges):
    mesh = plsc.VectorSubcoreMesh(core_axis_name="c", subcore_axis_name="s")
    nc, ns = mesh.num_cores, mesh.num_subcores
    nw = nc * ns
    ep = n_edges // nw
    nchunk = ep // _L

    @functools.partial(
        pl.kernel,
        out_type=jax.ShapeDtypeStruct((nw, _L), jnp.float32),
        mesh=mesh,
        compiler_params=pltpu.CompilerParams(needs_layout_passes=False),
        scratch_types=[
            pltpu.VMEM((n_nodes,), jnp.int32),
            pltpu.VMEM((ep,), jnp.int32),
            pltpu.VMEM((ep,), jnp.float32),
            pltpu.VMEM((_L * _L,), jnp.float32),
            pltpu.VMEM((_L,), jnp.float32),
        ],
    )
    def sc_min(src_hbm, batch_hbm, attr_hbm, out_hbm,
               batch_v, src_v, attr_v, minacc, minout):
        wid = lax.axis_index("s") * nc + lax.axis_index("c")
        base = pl.multiple_of(wid * ep, 8)
        pltpu.sync_copy(batch_hbm, batch_v)
        pltpu.sync_copy(src_hbm.at[pl.ds(base, ep)], src_v)
        pltpu.sync_copy(attr_hbm.at[pl.ds(base, ep)], attr_v)

        for l in range(_L):
            minacc[pl.ds(l * _L, _L)] = jnp.full((_L,), _BIG, jnp.float32)

        lane16 = lax.iota(jnp.int32, _L) * _L

        def body(c, carry):
            b = pl.multiple_of(c * _L, 8)
            sv = src_v[pl.ds(b, _L)]
            eb = plsc.load_gather(batch_v, [sv])
            av = attr_v[pl.ds(b, _L)]
            idx = lane16 + eb
            cur = plsc.load_gather(minacc, [idx])
            plsc.store_scatter(minacc, [idx], jnp.minimum(cur, av))
            return carry

        lax.fori_loop(0, nchunk, body, 0)

        m = minacc[pl.ds(0, _L)]
        for l in range(1, _L):
            m = jnp.minimum(m, minacc[pl.ds(l * _L, _L)])
        minout[...] = m
        pltpu.sync_copy(minout, out_hbm.at[wid])

    return sc_min


def _make_sc_edge(n_nodes, n_edges, d, n_pad):
    mesh = plsc.VectorSubcoreMesh(core_axis_name="c", subcore_axis_name="s")
    nc, ns = mesh.num_cores, mesh.num_subcores
    nw = nc * ns
    ep = n_edges // nw
    eblk = 2000
    nblk = ep // eblk
    nchunk = eblk // _L
    rows_per_tile = n_pad // ns
    zrows = 64
    nvec = d // _L

    @functools.partial(
        pl.kernel,
        out_type=jax.ShapeDtypeStruct((nc, n_pad, d), jnp.float32),
        mesh=mesh,
        compiler_params=pltpu.CompilerParams(needs_layout_passes=False),
        scratch_types=[
            pltpu.VMEM((n_nodes,), jnp.int32),
            pltpu.VMEM((eblk,), jnp.int32),
            pltpu.VMEM((eblk,), jnp.int32),
            pltpu.VMEM((eblk,), jnp.float32),
            pltpu.VMEM((nw, _L), jnp.float32),
            pltpu.VMEM((_L,), jnp.float32),
            pltpu.VMEM((_L,), jnp.float32),
            pltpu.VMEM((_L,), jnp.int32),
            pltpu.VMEM((_L,), jnp.int32),
            pltpu.VMEM((_L, d), jnp.float32),
            pltpu.VMEM((zrows, d), jnp.float32),
            pltpu.VMEM_SHARED((n_pad, d), jnp.float32),
            pltpu.SemaphoreType.DMA,
        ],
    )
    def sc_edge(src_hbm, dst_hbm, attr_hbm, batch_hbm, h_hbm, mins_hbm,
                out_hbm,
                batch_v, src_v, dst_v, attr_v, mins_v, min_buf, ea_buf,
                idx_buf, didx_buf, rows, zbuf, agg_sh, sem):
        cid = lax.axis_index("c")
        sid = lax.axis_index("s")
        wid = sid * nc + cid
        base = pl.multiple_of(wid * ep, 8)
        row0 = sid * rows_per_tile

        def zb(i, carry):
            for j in range(nvec):
                zbuf[i, pl.ds(j * _L, _L)] = jnp.zeros((_L,), jnp.float32)
            return carry

        lax.fori_loop(0, zrows, zb, 0)
        for t in range(rows_per_tile // zrows):
            pltpu.sync_copy(zbuf, agg_sh.at[pl.ds(row0 + t * zrows, zrows)])
        plsc.subcore_barrier()

        pltpu.sync_copy(batch_hbm, batch_v)
        pltpu.sync_copy(mins_hbm, mins_v)

        m = mins_v[0]
        for r in range(1, nw):
            m = jnp.minimum(m, mins_v[r])
        min_buf[...] = m

        def body(c, carry):
            b = pl.multiple_of(c * _L, 8)
            sv = src_v[pl.ds(b, _L)]
            idx_buf[...] = sv
            eb = plsc.load_gather(batch_v, [sv])
            mv = plsc.load_gather(min_buf, [eb])
            av = attr_v[pl.ds(b, _L)]
            dd = av - mv
            ea_buf[...] = jnp.exp(dd * dd * (-_INV_S))
            didx_buf[...] = dst_v[pl.ds(b, _L)]
            pltpu.async_copy(h_hbm.at[idx_buf], rows, sem).wait()
            for k in range(_L):
                s = plsc.load_gather(ea_buf, [jnp.full((_L,), k, jnp.int32)])
                for j in range(nvec):
                    rows[k, pl.ds(j * _L, _L)] = rows[k, pl.ds(j * _L, _L)] * s
            pltpu.sync_copy(rows, agg_sh.at[didx_buf], add=True)
            return carry

        for blk in range(nblk):
            eb0 = pl.multiple_of(base + blk * eblk, 8)
            pltpu.sync_copy(src_hbm.at[pl.ds(eb0, eblk)], src_v)
            pltpu.sync_copy(dst_hbm.at[pl.ds(eb0, eblk)], dst_v)
            pltpu.sync_copy(attr_hbm.at[pl.ds(eb0, eblk)], attr_v)
            lax.fori_loop(0, nchunk, body, 0)

        plsc.subcore_barrier()

        for t in range(rows_per_tile // zrows):
            r = row0 + t * zrows
            pltpu.sync_copy(agg_sh.at[pl.ds(r, zrows)], zbuf)
            pltpu.sync_copy(zbuf, out_hbm.at[cid, pl.ds(r, zrows)])

    return sc_edge


def _pool_body(nb, aggs_ref, batch_ref, b1_ref, macro_ref, mean_ref, std_ref,
               o_ref, pooled, cnt):
    i = pl.program_id(0)

    @pl.when(i == 0)
    def _():
        pooled[...] = jnp.zeros_like(pooled)
        cnt[...] = jnp.zeros_like(cnt)

    a = aggs_ref[0] + aggs_ref[1]
    emb = jnp.maximum(a + b1_ref[...], 0.0)
    gids = lax.broadcasted_iota(jnp.int32, (1, 16), 1)
    mask = (batch_ref[...] == gids).astype(jnp.float32)
    pooled[...] += lax.dot_general(
        mask, emb, (((0,), (0,)), ((), ())),
        preferred_element_type=jnp.float32, precision=lax.Precision.HIGHEST)
    cnt[...] += jnp.sum(mask, axis=0, keepdims=True)

    @pl.when(i == nb - 1)
    def _():
        gemb = pooled[...] / jnp.maximum(cnt[...], 1.0).reshape(16, 1)
        tm = (macro_ref[...] - mean_ref[...]) / (std_ref[...] + 1e-06)
        o_ref[...] = jnp.concatenate([gemb, tm], axis=1)


def _pool(aggs, batch2d, b1, macro, mean, std, nb):
    _, n, d = aggs.shape
    g, dm = macro.shape
    blk = n // nb
    return pl.pallas_call(
        functools.partial(_pool_body, nb),
        grid=(nb,),
        in_specs=[
            pl.BlockSpec((2, blk, d), lambda i: (0, i, 0)),
            pl.BlockSpec((blk, 1), lambda i: (i, 0)),
            pl.BlockSpec((1, d), lambda i: (0, 0)),
            pl.BlockSpec((g, dm), lambda i: (0, 0)),
            pl.BlockSpec((1, dm), lambda i: (0, 0)),
            pl.BlockSpec((1, dm), lambda i: (0, 0)),
        ],
        out_specs=pl.BlockSpec((g, d + dm), lambda i: (0, 0)),
        out_shape=jax.ShapeDtypeStruct((g, d + dm), jnp.float32),
        scratch_shapes=[
            pltpu.VMEM((g, d), jnp.float32),
            pltpu.VMEM((1, g), jnp.float32),
        ],
    )(aggs, batch2d, b1, macro, mean, std)


def kernel(x, edge_index, edge_attr, batch, macro_metrics, W1, b1,
           macro_mean, macro_std):
    n, d = x.shape
    e = edge_attr.shape[0]
    g, dm = macro_metrics.shape
    src = edge_index[0]
    dst = edge_index[1]

    n_pad = ((n + 639) // 640) * 640

    h = _matmul(x, W1, nb=5)
    mins = _make_sc_min(n, e)(src, batch, edge_attr)
    aggs = _make_sc_edge(n, e, d, n_pad)(src, dst, edge_attr, batch, h, mins)
    batch_p = jnp.concatenate(
        [batch, jnp.full((n_pad - n,), g, jnp.int32)]).reshape(n_pad, 1)
    out = _pool(aggs, batch_p, b1.reshape(1, d), macro_metrics,
                macro_mean.reshape(1, dm), macro_std.reshape(1, dm), nb=5)
    return out


# speedup vs baseline: 7.1314x; 1.0761x over previous
import functools

import jax
import jax.numpy as jnp
from jax import lax
from jax.experimental import pallas as pl
from jax.experimental.pallas import tpu as pltpu
from jax.experimental.pallas import tpu_sc as plsc

_SIGMA = 1.0
_INV_S = 1.0 / (_SIGMA**2 + 1e-06)
_BIG = 3.0e38
_L = 16


def _mm_body(x_ref, w_ref, o_ref):
    o_ref[...] = jnp.dot(x_ref[...], w_ref[...],
                         preferred_element_type=jnp.float32,
                         precision=lax.Precision.HIGHEST)


def _matmul(x, w, nb):
    n, d = x.shape
    blk = n // nb
    return pl.pallas_call(
        _mm_body,
        grid=(nb,),
        in_specs=[
            pl.BlockSpec((blk, d), lambda i: (i, 0)),
            pl.BlockSpec((d, d), lambda i: (0, 0)),
        ],
        out_specs=pl.BlockSpec((blk, d), lambda i: (i, 0)),
        out_shape=jax.ShapeDtypeStruct((n, d), jnp.float32),
    )(x, w)


def _make_sc_min(n_nodes, n_edges):
    mesh = plsc.VectorSubcoreMesh(core_axis_name="c", subcore_axis_name="s")
    nc, ns = mesh.num_cores, mesh.num_subcores
    nw = nc * ns
    ep = n_edges // nw
    nchunk = ep // _L

    @functools.partial(
        pl.kernel,
        out_type=jax.ShapeDtypeStruct((nw, _L), jnp.float32),
        mesh=mesh,
        compiler_params=pltpu.CompilerParams(needs_layout_passes=False),
        scratch_types=[
            pltpu.VMEM((n_nodes,), jnp.int32),
            pltpu.VMEM((ep,), jnp.int32),
            pltpu.VMEM((ep,), jnp.float32),
            pltpu.VMEM((_L * _L,), jnp.float32),
            pltpu.VMEM((_L,), jnp.float32),
        ],
    )
    def sc_min(src_hbm, batch_hbm, attr_hbm, out_hbm,
               batch_v, src_v, attr_v, minacc, minout):
        wid = lax.axis_index("s") * nc + lax.axis_index("c")
        base = pl.multiple_of(wid * ep, 8)
        pltpu.sync_copy(batch_hbm, batch_v)
        pltpu.sync_copy(src_hbm.at[pl.ds(base, ep)], src_v)
        pltpu.sync_copy(attr_hbm.at[pl.ds(base, ep)], attr_v)

        for l in range(_L):
            minacc[pl.ds(l * _L, _L)] = jnp.full((_L,), _BIG, jnp.float32)

        lane16 = lax.iota(jnp.int32, _L) * _L

        def body(c, carry):
            b = pl.multiple_of(c * _L, 8)
            sv = src_v[pl.ds(b, _L)]
            eb = plsc.load_gather(batch_v, [sv])
            av = attr_v[pl.ds(b, _L)]
            idx = lane16 + eb
            cur = plsc.load_gather(minacc, [idx])
            plsc.store_scatter(minacc, [idx], jnp.minimum(cur, av))
            return carry

        lax.fori_loop(0, nchunk, body, 0)

        m = minacc[pl.ds(0, _L)]
        for l in range(1, _L):
            m = jnp.minimum(m, minacc[pl.ds(l * _L, _L)])
        minout[...] = m
        pltpu.sync_copy(minout, out_hbm.at[wid])

    return sc_min


def _make_sc_edge(n_nodes, n_edges, d, n_pad):
    mesh = plsc.VectorSubcoreMesh(core_axis_name="c", subcore_axis_name="s")
    nc, ns = mesh.num_cores, mesh.num_subcores
    nw = nc * ns
    ep = n_edges // nw
    eblk = 2048
    nblk = ep // eblk
    ck = 64
    nchunk = eblk // ck
    rows_per_tile = n_pad // ns
    zrows = 32
    nvec = d // _L

    @functools.partial(
        pl.kernel,
        out_type=jax.ShapeDtypeStruct((nc, n_pad, d), jnp.float32),
        mesh=mesh,
        compiler_params=pltpu.CompilerParams(needs_layout_passes=False),
        scratch_types=[
            pltpu.VMEM((n_nodes,), jnp.int32),
            pltpu.VMEM((eblk,), jnp.int32),
            pltpu.VMEM((eblk,), jnp.int32),
            pltpu.VMEM((eblk,), jnp.float32),
            pltpu.VMEM((nw, _L), jnp.float32),
            pltpu.VMEM((_L,), jnp.float32),
            [pltpu.VMEM((ck,), jnp.float32) for _ in range(2)],
            [pltpu.VMEM((ck,), jnp.int32) for _ in range(2)],
            [pltpu.VMEM((ck,), jnp.int32) for _ in range(2)],
            [pltpu.VMEM((ck, d), jnp.float32) for _ in range(2)],
            pltpu.VMEM((zrows, d), jnp.float32),
            pltpu.VMEM_SHARED((n_pad, d), jnp.float32),
            [pltpu.SemaphoreType.DMA for _ in range(2)],
        ],
    )
    def sc_edge(src_hbm, dst_hbm, attr_hbm, batch_hbm, h_hbm, mins_hbm,
                out_hbm,
                batch_v, src_v, dst_v, attr_v, mins_v, min_buf, ea_buf,
                idx_buf, didx_buf, rows, zbuf, agg_sh, sems):
        cid = lax.axis_index("c")
        sid = lax.axis_index("s")
        wid = sid * nc + cid
        base = pl.multiple_of(wid * ep, 8)
        row0 = sid * rows_per_tile

        def zb(i, carry):
            for j in range(nvec):
                zbuf[i, pl.ds(j * _L, _L)] = jnp.zeros((_L,), jnp.float32)
            return carry

        lax.fori_loop(0, zrows, zb, 0)
        for t in range(rows_per_tile // zrows):
            pltpu.sync_copy(zbuf, agg_sh.at[pl.ds(row0 + t * zrows, zrows)])
        plsc.subcore_barrier()

        pltpu.sync_copy(batch_hbm, batch_v)
        pltpu.sync_copy(mins_hbm, mins_v)

        m = mins_v[0]
        for r in range(1, nw):
            m = jnp.minimum(m, mins_v[r])
        min_buf[...] = m

        def meta_fire(c, p):
            b = pl.multiple_of(c * ck, 8)
            for q in range(ck // _L):
                sv = src_v[pl.ds(b + q * _L, _L)]
                idx_buf[p][pl.ds(q * _L, _L)] = sv
                eb = plsc.load_gather(batch_v, [sv])
                mv = plsc.load_gather(min_buf, [eb])
                av = attr_v[pl.ds(b + q * _L, _L)]
                dd = av - mv
                ea_buf[p][pl.ds(q * _L, _L)] = jnp.exp(dd * dd * (-_INV_S))
                didx_buf[p][pl.ds(q * _L, _L)] = dst_v[pl.ds(b + q * _L, _L)]
            pltpu.async_copy(h_hbm.at[idx_buf[p]], rows[p], sems[p])

        def consume(c, p):
            pltpu.make_async_copy(h_hbm.at[idx_buf[p]], rows[p],
                                  sems[p]).wait()
            for k in range(ck):
                s = plsc.load_gather(ea_buf[p],
                                     [jnp.full((_L,), k, jnp.int32)])
                for j in range(nvec):
                    rows[p][k, pl.ds(j * _L, _L)] = (
                        rows[p][k, pl.ds(j * _L, _L)] * s)
            pltpu.sync_copy(rows[p], agg_sh.at[didx_buf[p]], add=True)

        def block(blk, carry):
            eb0 = pl.multiple_of(base + blk * eblk, 8)
            pltpu.sync_copy(src_hbm.at[pl.ds(eb0, eblk)], src_v)
            pltpu.sync_copy(dst_hbm.at[pl.ds(eb0, eblk)], dst_v)
            pltpu.sync_copy(attr_hbm.at[pl.ds(eb0, eblk)], attr_v)
            meta_fire(0, 0)
            meta_fire(1, 1)

            def pair(t, c2):
                for p in range(2):
                    c = 2 * t + p
                    consume(c, p)

                    @pl.when(c + 2 < nchunk)
                    def _():
                        meta_fire(c + 2, p)
                return c2

            lax.fori_loop(0, nchunk // 2, pair, 0)
            return carry

        lax.fori_loop(0, nblk, block, 0)

        plsc.subcore_barrier()

        for t in range(rows_per_tile // zrows):
            r = row0 + t * zrows
            pltpu.sync_copy(agg_sh.at[pl.ds(r, zrows)], zbuf)
            pltpu.sync_copy(zbuf, out_hbm.at[cid, pl.ds(r, zrows)])

    return sc_edge


def _pool_body(nb, aggs_ref, batch_ref, b1_ref, macro_ref, mean_ref, std_ref,
               o_ref, pooled, cnt):
    i = pl.program_id(0)

    @pl.when(i == 0)
    def _():
        pooled[...] = jnp.zeros_like(pooled)
        cnt[...] = jnp.zeros_like(cnt)

    a = aggs_ref[0] + aggs_ref[1]
    emb = jnp.maximum(a + b1_ref[...], 0.0)
    gids = lax.broadcasted_iota(jnp.int32, (1, 16), 1)
    mask = (batch_ref[...] == gids).astype(jnp.float32)
    pooled[...] += lax.dot_general(
        mask, emb, (((0,), (0,)), ((), ())),
        preferred_element_type=jnp.float32, precision=lax.Precision.HIGHEST)
    cnt[...] += jnp.sum(mask, axis=0, keepdims=True)

    @pl.when(i == nb - 1)
    def _():
        gemb = pooled[...] / jnp.maximum(cnt[...], 1.0).reshape(16, 1)
        tm = (macro_ref[...] - mean_ref[...]) / (std_ref[...] + 1e-06)
        o_ref[...] = jnp.concatenate([gemb, tm], axis=1)


def _pool(aggs, batch2d, b1, macro, mean, std, nb):
    _, n, d = aggs.shape
    g, dm = macro.shape
    blk = n // nb
    return pl.pallas_call(
        functools.partial(_pool_body, nb),
        grid=(nb,),
        in_specs=[
            pl.BlockSpec((2, blk, d), lambda i: (0, i, 0)),
            pl.BlockSpec((blk, 1), lambda i: (i, 0)),
            pl.BlockSpec((1, d), lambda i: (0, 0)),
            pl.BlockSpec((g, dm), lambda i: (0, 0)),
            pl.BlockSpec((1, dm), lambda i: (0, 0)),
            pl.BlockSpec((1, dm), lambda i: (0, 0)),
        ],
        out_specs=pl.BlockSpec((g, d + dm), lambda i: (0, 0)),
        out_shape=jax.ShapeDtypeStruct((g, d + dm), jnp.float32),
        scratch_shapes=[
            pltpu.VMEM((g, d), jnp.float32),
            pltpu.VMEM((1, g), jnp.float32),
        ],
    )(aggs, batch2d, b1, macro, mean, std)


def kernel(x, edge_index, edge_attr, batch, macro_metrics, W1, b1,
           macro_mean, macro_std):
    n, d = x.shape
    e = edge_attr.shape[0]
    g, dm = macro_metrics.shape
    src = edge_index[0]
    dst = edge_index[1]

    n_pad = ((n + 639) // 640) * 640
    e_pad = ((e + 65535) // 65536) * 65536
    if e_pad != e:
        pe = e_pad - e
        src = jnp.concatenate([src, jnp.zeros((pe,), jnp.int32)])
        dst = jnp.concatenate([dst, jnp.full((pe,), n_pad - 1, jnp.int32)])
        edge_attr = jnp.concatenate(
            [edge_attr, jnp.full((pe,), _BIG, jnp.float32)])

    h = _matmul(x, W1, nb=5)
    mins = _make_sc_min(n, e_pad)(src, batch, edge_attr)
    aggs = _make_sc_edge(n, e_pad, d, n_pad)(src, dst, edge_attr, batch, h,
                                             mins)
    batch_p = jnp.concatenate(
        [batch, jnp.full((n_pad - n,), g, jnp.int32)]).reshape(n_pad, 1)
    out = _pool(aggs, batch_p, b1.reshape(1, d), macro_metrics,
                macro_mean.reshape(1, dm), macro_std.reshape(1, dm), nb=5)
    return out


# speedup vs baseline: 7.5528x; 1.0591x over previous
import functools

import jax
import jax.numpy as jnp
from jax import lax
from jax.experimental import pallas as pl
from jax.experimental.pallas import tpu as pltpu
from jax.experimental.pallas import tpu_sc as plsc

_SIGMA = 1.0
_INV_S = 1.0 / (_SIGMA**2 + 1e-06)
_BIG = 3.0e38
_L = 16


def _mm_body(x_ref, w_ref, o_ref):
    o_ref[...] = jnp.dot(x_ref[...], w_ref[...],
                         preferred_element_type=jnp.float32,
                         precision=lax.Precision.HIGHEST)


def _matmul(x, w, nb):
    n, d = x.shape
    blk = n // nb
    return pl.pallas_call(
        _mm_body,
        grid=(nb,),
        in_specs=[
            pl.BlockSpec((blk, d), lambda i: (i, 0)),
            pl.BlockSpec((d, d), lambda i: (0, 0)),
        ],
        out_specs=pl.BlockSpec((blk, d), lambda i: (i, 0)),
        out_shape=jax.ShapeDtypeStruct((n, d), jnp.float32),
    )(x, w)


def _make_sc_min(n_nodes, n_edges):
    mesh = plsc.VectorSubcoreMesh(core_axis_name="c", subcore_axis_name="s")
    nc, ns = mesh.num_cores, mesh.num_subcores
    nw = nc * ns
    ep = n_edges // nw
    nchunk = ep // _L

    @functools.partial(
        pl.kernel,
        out_type=jax.ShapeDtypeStruct((nw, _L), jnp.float32),
        mesh=mesh,
        compiler_params=pltpu.CompilerParams(needs_layout_passes=False),
        scratch_types=[
            pltpu.VMEM((n_nodes,), jnp.int32),
            pltpu.VMEM((ep,), jnp.int32),
            pltpu.VMEM((ep,), jnp.float32),
            pltpu.VMEM((_L * _L,), jnp.float32),
            pltpu.VMEM((_L,), jnp.float32),
        ],
    )
    def sc_min(src_hbm, batch_hbm, attr_hbm, out_hbm,
               batch_v, src_v, attr_v, minacc, minout):
        wid = lax.axis_index("s") * nc + lax.axis_index("c")
        base = pl.multiple_of(wid * ep, 8)
        pltpu.sync_copy(batch_hbm, batch_v)
        pltpu.sync_copy(src_hbm.at[pl.ds(base, ep)], src_v)
        pltpu.sync_copy(attr_hbm.at[pl.ds(base, ep)], attr_v)

        for l in range(_L):
            minacc[pl.ds(l * _L, _L)] = jnp.full((_L,), _BIG, jnp.float32)

        lane16 = lax.iota(jnp.int32, _L) * _L

        def body(c, carry):
            b = pl.multiple_of(c * _L, 8)
            sv = src_v[pl.ds(b, _L)]
            eb = plsc.load_gather(batch_v, [sv])
            av = attr_v[pl.ds(b, _L)]
            idx = lane16 + eb
            cur = plsc.load_gather(minacc, [idx])
            plsc.store_scatter(minacc, [idx], jnp.minimum(cur, av))
            return carry

        lax.fori_loop(0, nchunk, body, 0)

        m = minacc[pl.ds(0, _L)]
        for l in range(1, _L):
            m = jnp.minimum(m, minacc[pl.ds(l * _L, _L)])
        minout[...] = m
        pltpu.sync_copy(minout, out_hbm.at[wid])

    return sc_min


def _make_sc_edge(n_nodes, n_edges, d, n_pad):
    mesh = plsc.VectorSubcoreMesh(core_axis_name="c", subcore_axis_name="s")
    nc, ns = mesh.num_cores, mesh.num_subcores
    nw = nc * ns
    ep = n_edges // nw
    eblk = 2048
    nblk = ep // eblk
    ck = 64
    nchunk = eblk // ck
    rows_per_tile = n_pad // ns
    zrows = 32
    nvec = d // _L

    @functools.partial(
        pl.kernel,
        out_type=jax.ShapeDtypeStruct((nc, n_pad, d), jnp.float32),
        mesh=mesh,
        compiler_params=pltpu.CompilerParams(needs_layout_passes=False),
        scratch_types=[
            pltpu.VMEM((n_nodes,), jnp.int32),
            pltpu.VMEM((eblk,), jnp.int32),
            pltpu.VMEM((eblk,), jnp.int32),
            pltpu.VMEM((eblk,), jnp.float32),
            pltpu.VMEM((nw, _L), jnp.float32),
            pltpu.VMEM((_L,), jnp.float32),
            [pltpu.VMEM((ck,), jnp.float32) for _ in range(2)],
            [pltpu.VMEM((ck,), jnp.int32) for _ in range(2)],
            [pltpu.VMEM((ck,), jnp.int32) for _ in range(2)],
            [pltpu.VMEM((ck, d), jnp.float32) for _ in range(2)],
            pltpu.VMEM((zrows, d), jnp.float32),
            pltpu.VMEM_SHARED((n_pad, d), jnp.float32),
            [pltpu.SemaphoreType.DMA for _ in range(2)],
        ],
    )
    def sc_edge(src_hbm, dst_hbm, attr_hbm, batch_hbm, h_hbm, mins_hbm,
                out_hbm,
                batch_v, src_v, dst_v, attr_v, mins_v, min_buf, ea_buf,
                idx_buf, didx_buf, rows, zbuf, agg_sh, sems):
        cid = lax.axis_index("c")
        sid = lax.axis_index("s")
        wid = sid * nc + cid
        base = pl.multiple_of(wid * ep, 8)
        row0 = sid * rows_per_tile

        def zb(i, carry):
            for j in range(nvec):
                zbuf[i, pl.ds(j * _L, _L)] = jnp.zeros((_L,), jnp.float32)
            return carry

        lax.fori_loop(0, zrows, zb, 0)
        for t in range(rows_per_tile // zrows):
            pltpu.sync_copy(zbuf, agg_sh.at[pl.ds(row0 + t * zrows, zrows)])
        plsc.subcore_barrier()

        pltpu.sync_copy(batch_hbm, batch_v)
        pltpu.sync_copy(mins_hbm, mins_v)

        m = mins_v[0]
        for r in range(1, nw):
            m = jnp.minimum(m, mins_v[r])
        min_buf[...] = m

        def meta_fire(c, p):
            b = pl.multiple_of(c * ck, 8)
            for q in range(ck // _L):
                sv = src_v[pl.ds(b + q * _L, _L)]
                idx_buf[p][pl.ds(q * _L, _L)] = sv
                eb = plsc.load_gather(batch_v, [sv])
                mv = plsc.load_gather(min_buf, [eb])
                av = attr_v[pl.ds(b + q * _L, _L)]
                dd = av - mv
                ea_buf[p][pl.ds(q * _L, _L)] = jnp.exp(dd * dd * (-_INV_S))
                didx_buf[p][pl.ds(q * _L, _L)] = dst_v[pl.ds(b + q * _L, _L)]
            pltpu.async_copy(h_hbm.at[idx_buf[p]], rows[p], sems[p])

        def consume(c, p):
            pltpu.make_async_copy(h_hbm.at[idx_buf[p]], rows[p],
                                  sems[p]).wait()

            @plsc.parallel_loop(0, ck, 1, unroll=4)
            def _(k):
                s = plsc.load_gather(ea_buf[p], [jnp.full((_L,), 1, jnp.int32) * k])
                for j in range(nvec):
                    rows[p][k, pl.ds(j * _L, _L)] = (
                        rows[p][k, pl.ds(j * _L, _L)] * s)

            pltpu.sync_copy(rows[p], agg_sh.at[didx_buf[p]], add=True)

        def block(blk, carry):
            eb0 = pl.multiple_of(base + blk * eblk, 8)
            pltpu.sync_copy(src_hbm.at[pl.ds(eb0, eblk)], src_v)
            pltpu.sync_copy(dst_hbm.at[pl.ds(eb0, eblk)], dst_v)
            pltpu.sync_copy(attr_hbm.at[pl.ds(eb0, eblk)], attr_v)
            meta_fire(0, 0)
            meta_fire(1, 1)

            def pair(t, c2):
                for p in range(2):
                    c = 2 * t + p
                    consume(c, p)

                    @pl.when(c + 2 < nchunk)
                    def _():
                        meta_fire(c + 2, p)
                return c2

            lax.fori_loop(0, nchunk // 2, pair, 0)
            return carry

        lax.fori_loop(0, nblk, block, 0)

        plsc.subcore_barrier()

        for t in range(rows_per_tile // zrows):
            r = row0 + t * zrows
            pltpu.sync_copy(agg_sh.at[pl.ds(r, zrows)], zbuf)
            pltpu.sync_copy(zbuf, out_hbm.at[cid, pl.ds(r, zrows)])

    return sc_edge


def _pool_body(nb, aggs_ref, batch_ref, b1_ref, macro_ref, mean_ref, std_ref,
               o_ref, pooled, cnt):
    i = pl.program_id(0)

    @pl.when(i == 0)
    def _():
        pooled[...] = jnp.zeros_like(pooled)
        cnt[...] = jnp.zeros_like(cnt)

    a = aggs_ref[0] + aggs_ref[1]
    emb = jnp.maximum(a + b1_ref[...], 0.0)
    gids = lax.broadcasted_iota(jnp.int32, (1, 16), 1)
    mask = (batch_ref[...] == gids).astype(jnp.float32)
    pooled[...] += lax.dot_general(
        mask, emb, (((0,), (0,)), ((), ())),
        preferred_element_type=jnp.float32, precision=lax.Precision.HIGHEST)
    cnt[...] += jnp.sum(mask, axis=0, keepdims=True)

    @pl.when(i == nb - 1)
    def _():
        gemb = pooled[...] / jnp.maximum(cnt[...], 1.0).reshape(16, 1)
        tm = (macro_ref[...] - mean_ref[...]) / (std_ref[...] + 1e-06)
        o_ref[...] = jnp.concatenate([gemb, tm], axis=1)


def _pool(aggs, batch2d, b1, macro, mean, std, nb):
    _, n, d = aggs.shape
    g, dm = macro.shape
    blk = n // nb
    return pl.pallas_call(
        functools.partial(_pool_body, nb),
        grid=(nb,),
        in_specs=[
            pl.BlockSpec((2, blk, d), lambda i: (0, i, 0)),
            pl.BlockSpec((blk, 1), lambda i: (i, 0)),
            pl.BlockSpec((1, d), lambda i: (0, 0)),
            pl.BlockSpec((g, dm), lambda i: (0, 0)),
            pl.BlockSpec((1, dm), lambda i: (0, 0)),
            pl.BlockSpec((1, dm), lambda i: (0, 0)),
        ],
        out_specs=pl.BlockSpec((g, d + dm), lambda i: (0, 0)),
        out_shape=jax.ShapeDtypeStruct((g, d + dm), jnp.float32),
        scratch_shapes=[
            pltpu.VMEM((g, d), jnp.float32),
            pltpu.VMEM((1, g), jnp.float32),
        ],
    )(aggs, batch2d, b1, macro, mean, std)


def kernel(x, edge_index, edge_attr, batch, macro_metrics, W1, b1,
           macro_mean, macro_std):
    n, d = x.shape
    e = edge_attr.shape[0]
    g, dm = macro_metrics.shape
    src = edge_index[0]
    dst = edge_index[1]

    n_pad = ((n + 639) // 640) * 640
    e_pad = ((e + 65535) // 65536) * 65536
    if e_pad != e:
        pe = e_pad - e
        src = jnp.concatenate([src, jnp.zeros((pe,), jnp.int32)])
        dst = jnp.concatenate([dst, jnp.full((pe,), n_pad - 1, jnp.int32)])
        edge_attr = jnp.concatenate(
            [edge_attr, jnp.full((pe,), _BIG, jnp.float32)])

    h = _matmul(x, W1, nb=5)
    mins = _make_sc_min(n, e_pad)(src, batch, edge_attr)
    aggs = _make_sc_edge(n, e_pad, d, n_pad)(src, dst, edge_attr, batch, h,
                                             mins)
    batch_p = jnp.concatenate(
        [batch, jnp.full((n_pad - n,), g, jnp.int32)]).reshape(n_pad, 1)
    out = _pool(aggs, batch_p, b1.reshape(1, d), macro_metrics,
                macro_mean.reshape(1, dm), macro_std.reshape(1, dm), nb=5)
    return out
